# baseline (speedup 1.0000x reference)
import functools

import jax
import jax.numpy as jnp
from jax import lax
from jax.experimental import pallas as pl
from jax.experimental.pallas import tpu as pltpu
from jax.experimental.pallas import tpu_sc as plsc

_CHUNK = 64


def _make_sc_kernel(batch, seq, dim, dtype):
    info = plsc.get_sparse_core_info()
    nw = info.num_cores * info.num_subcores
    rows_per_w = seq // nw
    nch = rows_per_w // _CHUNK
    mesh = plsc.VectorSubcoreMesh(core_axis_name="c", subcore_axis_name="s")

    @functools.partial(
        pl.kernel,
        mesh=mesh,
        out_type=jax.ShapeDtypeStruct((batch, seq, dim), dtype),
        scratch_types=[
            pltpu.VMEM((nch, _CHUNK, dim), dtype),
            pltpu.SemaphoreType.DMA((nch,)),
            pltpu.SemaphoreType.DMA,
        ],
    )
    def k(pe_hbm, out_hbm, buf, rsems, wsem):
        wid = lax.axis_index("s") * info.num_cores + lax.axis_index("c")
        base = wid * rows_per_w
        reads = []
        for c in range(nch):
            r = pltpu.make_async_copy(
                pe_hbm.at[pl.ds(base + c * _CHUNK, _CHUNK)],
                buf.at[c],
                rsems.at[c],
            )
            r.start()
            reads.append(r)
        writes = []
        for c in range(nch):
            reads[c].wait()
            for b in range(batch):
                w = pltpu.make_async_copy(
                    buf.at[c],
                    out_hbm.at[b, pl.ds(base + c * _CHUNK, _CHUNK)],
                    wsem,
                )
                w.start()
                writes.append(w)
        for w in writes:
            w.wait()

    return k


def kernel(mask, pe):
    batch, seq = mask.shape
    max_len, dim = pe.shape
    k = _make_sc_kernel(batch, seq, dim, pe.dtype)
    return k(pe[:seq])


# speedup vs baseline: 1.7377x; 1.7377x over previous
import jax
import jax.numpy as jnp
from jax.experimental import pallas as pl
from jax.experimental.pallas import tpu as pltpu

_NCH = 32


def _body(pe_hbm, out_hbm, buf, rsems, wsem):
    batch = out_hbm.shape[0]
    seq = pe_hbm.shape[0]
    ch = seq // _NCH
    reads = []
    for c in range(_NCH):
        r = pltpu.make_async_copy(
            pe_hbm.at[pl.ds(c * ch, ch)], buf.at[pl.ds(c * ch, ch)], rsems.at[c]
        )
        r.start()
        reads.append(r)
    writes = []
    for c in range(_NCH):
        reads[c].wait()
        for b in range(batch):
            w = pltpu.make_async_copy(
                buf.at[pl.ds(c * ch, ch)], out_hbm.at[b, pl.ds(c * ch, ch)], wsem
            )
            w.start()
            writes.append(w)
    for w in writes:
        w.wait()


def kernel(mask, pe):
    batch, seq = mask.shape
    max_len, dim = pe.shape
    out = pl.pallas_call(
        _body,
        in_specs=[pl.BlockSpec(memory_space=pltpu.HBM)],
        out_specs=pl.BlockSpec(memory_space=pltpu.HBM),
        out_shape=jax.ShapeDtypeStruct((batch, seq, dim), pe.dtype),
        scratch_shapes=[
            pltpu.VMEM((seq, dim), pe.dtype),
            pltpu.SemaphoreType.DMA((_NCH,)),
            pltpu.SemaphoreType.DMA,
        ],
    )(pe[:seq])
    return out


# speedup vs baseline: 1.7590x; 1.0122x over previous
import jax
import jax.numpy as jnp
from jax.experimental import pallas as pl
from jax.experimental.pallas import tpu as pltpu

_NCH = 8


def _body(pe_hbm, out_hbm, buf, rsems, wsem):
    batch = out_hbm.shape[0]
    seq = pe_hbm.shape[0]
    ch = seq // _NCH
    reads = []
    for c in range(_NCH):
        r = pltpu.make_async_copy(
            pe_hbm.at[pl.ds(c * ch, ch)], buf.at[pl.ds(c * ch, ch)], rsems.at[c]
        )
        r.start()
        reads.append(r)
    writes = []
    for c in range(_NCH):
        reads[c].wait()
        for b in range(batch):
            w = pltpu.make_async_copy(
                buf.at[pl.ds(c * ch, ch)], out_hbm.at[b, pl.ds(c * ch, ch)], wsem
            )
            w.start()
            writes.append(w)
    for w in writes:
        w.wait()


def kernel(mask, pe):
    batch, seq = mask.shape
    max_len, dim = pe.shape
    out = pl.pallas_call(
        _body,
        in_specs=[pl.BlockSpec(memory_space=pltpu.HBM)],
        out_specs=pl.BlockSpec(memory_space=pltpu.HBM),
        out_shape=jax.ShapeDtypeStruct((batch, seq, dim), pe.dtype),
        scratch_shapes=[
            pltpu.VMEM((seq, dim), pe.dtype),
            pltpu.SemaphoreType.DMA((_NCH,)),
            pltpu.SemaphoreType.DMA,
        ],
    )(pe[:seq])
    return out


# speedup vs baseline: 1.7833x; 1.0138x over previous
import jax
import jax.numpy as jnp
from jax.experimental import pallas as pl
from jax.experimental.pallas import tpu as pltpu

_NCH = 4


def _body(pe_hbm, out_hbm, buf, rsems, wsem):
    batch = out_hbm.shape[0]
    seq = pe_hbm.shape[0]
    ch = seq // _NCH
    reads = []
    for c in range(_NCH):
        r = pltpu.make_async_copy(
            pe_hbm.at[pl.ds(c * ch, ch)], buf.at[pl.ds(c * ch, ch)], rsems.at[c]
        )
        r.start()
        reads.append(r)
    writes = []
    for c in range(_NCH):
        reads[c].wait()
        for b in range(batch):
            w = pltpu.make_async_copy(
                buf.at[pl.ds(c * ch, ch)], out_hbm.at[b, pl.ds(c * ch, ch)], wsem
            )
            w.start()
            writes.append(w)
    for w in writes:
        w.wait()


def kernel(mask, pe):
    batch, seq = mask.shape
    max_len, dim = pe.shape
    out = pl.pallas_call(
        _body,
        in_specs=[pl.BlockSpec(memory_space=pltpu.HBM)],
        out_specs=pl.BlockSpec(memory_space=pltpu.HBM),
        out_shape=jax.ShapeDtypeStruct((batch, seq, dim), pe.dtype),
        scratch_shapes=[
            pltpu.VMEM((seq, dim), pe.dtype),
            pltpu.SemaphoreType.DMA((_NCH,)),
            pltpu.SemaphoreType.DMA,
        ],
    )(pe[:seq])
    return out
